# baseline (speedup 1.0000x reference)
import functools

import jax
import jax.numpy as jnp
from jax import lax
from jax.experimental import pallas as pl
from jax.experimental.pallas import tpu as pltpu
from jax.experimental.pallas import tpu_sc as plsc

_BETA = 0.5


def _minsq_kernel(bm, bk, n_k, x_sq_ref, w_sq_ref, x_ref, w2_ref, out_ref):
    x_sq = x_sq_ref[...]
    run_min = None
    for c in range(n_k):
        dotneg = lax.dot_general(
            x_ref[...], w2_ref[pl.ds(c * bk, bk), :],
            dimension_numbers=(((1,), (1,)), ((), ())),
            preferred_element_type=jnp.float32,
        )
        sq = x_sq[:, None] + w_sq_ref[pl.ds(c * bk, bk)][None, :] + dotneg
        cmin = jnp.min(sq, axis=1)
        run_min = cmin if c == 0 else jnp.minimum(run_min, cmin)
    out_ref[0, 0, :] = run_min


def _argbelow_kernel(bm, bk, n_k, x_sq_ref, w_sq_ref, hi_ref, x_ref, w2_ref,
                     tok_ref):
    x_sq = x_sq_ref[...]
    hi = hi_ref[...]
    best = None
    for c in range(n_k):
        dotneg = lax.dot_general(
            x_ref[...], w2_ref[pl.ds(c * bk, bk), :],
            dimension_numbers=(((1,), (1,)), ((), ())),
            preferred_element_type=jnp.float32,
        )
        sq = x_sq[:, None] + w_sq_ref[pl.ds(c * bk, bk)][None, :] + dotneg
        iota = lax.broadcasted_iota(jnp.int32, (bm, bk), 1)
        cand = jnp.min(jnp.where(sq <= hi[:, None], iota, 3 * bk),
                       axis=1) + c * bk
        best = cand if c == 0 else jnp.minimum(best, cand)
    tok_ref[0, 0, :] = best


def _tc_stage(kernel_body, extra_vec_inputs, out_dtype, flat_x, w2, x_sq,
              w_sq, bm, bk):
    m_total, d = flat_x.shape
    k_total = w2.shape[0]
    n_m = m_total // bm
    n_k = k_total // bk
    vec_specs = [pl.BlockSpec((bm,), lambda m: (m,))
                 for _ in range(len(extra_vec_inputs))]
    out3 = pl.pallas_call(
        functools.partial(kernel_body, bm, bk, n_k),
        grid=(n_m,),
        in_specs=[
            pl.BlockSpec((bm,), lambda m: (m,)),
            pl.BlockSpec((k_total,), lambda m: (0,)),
            *vec_specs,
            pl.BlockSpec((bm, d), lambda m: (m, 0)),
            pl.BlockSpec((k_total, d), lambda m: (0, 0)),
        ],
        out_specs=pl.BlockSpec((1, 1, bm), lambda m: (m, 0, 0)),
        out_shape=jax.ShapeDtypeStruct((n_m, 1, bm), out_dtype),
        compiler_params=pltpu.CompilerParams(
            vmem_limit_bytes=100 * 1024 * 1024,
        ),
    )(x_sq, w_sq, *extra_vec_inputs, flat_x, w2)
    return out3.reshape(m_total)


def _sc_gather(table, idx):
    v, d = table.shape
    m_total = idx.shape[0]
    nw = 32
    b_per_w = m_total // nw
    chunk = 192
    n_chunks = b_per_w // chunk
    mesh = plsc.VectorSubcoreMesh(core_axis_name="c", subcore_axis_name="s")

    @functools.partial(
        pl.kernel,
        mesh=mesh,
        out_type=jax.ShapeDtypeStruct((m_total, d), jnp.float32),
        scratch_types=[
            pltpu.VMEM((chunk,), jnp.int32),
            pltpu.VMEM((chunk, d), jnp.float32),
            pltpu.SemaphoreType.DMA,
        ],
    )
    def gather_kernel(table_hbm, idx_hbm, out_hbm, idx_v, rows_v, sem):
        wid = lax.axis_index("s") * 2 + lax.axis_index("c")
        base = wid * b_per_w
        for j in range(n_chunks):
            off = base + j * chunk
            pltpu.sync_copy(idx_hbm.at[pl.ds(off, chunk)], idx_v)
            pltpu.async_copy(table_hbm.at[idx_v], rows_v, sem).wait()
            pltpu.sync_copy(rows_v, out_hbm.at[pl.ds(off, chunk)])

    return gather_kernel(table, idx)


def kernel(x, w):
    b, n, d = x.shape
    m_total = b * n
    flat_x = x.reshape(-1, d)
    x_sq = jnp.sum(flat_x * flat_x, axis=1)
    w_sq = jnp.sum(w * w, axis=1)
    w2 = -2.0 * w

    min_sq = _tc_stage(_minsq_kernel, [], jnp.float32,
                       flat_x, w2, x_sq, w_sq, bm=1024, bk=2048)
    best_d = jnp.sqrt(jnp.maximum(min_sq, 0.0))
    v0i = lax.bitcast_convert_type(best_d * best_d, jnp.int32)
    hi = None
    for koff in range(-4, 5):
        vk = lax.bitcast_convert_type(jnp.maximum(v0i + koff, 0),
                                      jnp.float32)
        ok = jnp.sqrt(jnp.maximum(vk, 0.0)) == best_d
        cand = jnp.where(ok, vk, -jnp.inf)
        hi = cand if hi is None else jnp.maximum(hi, cand)

    tokens_flat = _tc_stage(_argbelow_kernel, [hi], jnp.int32,
                            flat_x, w2, x_sq, w_sq, bm=1024, bk=2048)
    vq_loss = jnp.sum(best_d * best_d) * ((1.0 + _BETA) / (m_total * d))

    quantized = _sc_gather(w, tokens_flat).reshape(b, n, d)
    tokens = tokens_flat.reshape(b, n)
    quantized_st = x + (quantized - x)
    return (tokens, quantized_st, vq_loss)


# speedup vs baseline: 1.5224x; 1.5224x over previous
import functools

import jax
import jax.numpy as jnp
from jax import lax
from jax.experimental import pallas as pl
from jax.experimental.pallas import tpu as pltpu
from jax.experimental.pallas import tpu_sc as plsc

_BETA = 0.5


def _minsq_kernel(bm, bk, n_k, x_sq_ref, w_sq_ref, x_ref, w2_ref, out_ref):
    x_sq = x_sq_ref[...]
    run_min = None
    for c in range(n_k):
        dotneg = lax.dot_general(
            x_ref[...], w2_ref[pl.ds(c * bk, bk), :],
            dimension_numbers=(((1,), (1,)), ((), ())),
            preferred_element_type=jnp.float32,
        )
        sq = x_sq[:, None] + w_sq_ref[pl.ds(c * bk, bk)][None, :] + dotneg
        cmin = jnp.min(sq, axis=1)
        run_min = cmin if c == 0 else jnp.minimum(run_min, cmin)
    out_ref[0, 0, :] = run_min


def _argbelow_kernel(bm, bk, n_k, x_sq_ref, w_sq_ref, hi_ref, x_ref, w2_ref,
                     tok_ref):
    x_sq = x_sq_ref[...]
    hi = hi_ref[...]
    best = None
    for c in range(n_k):
        dotneg = lax.dot_general(
            x_ref[...], w2_ref[pl.ds(c * bk, bk), :],
            dimension_numbers=(((1,), (1,)), ((), ())),
            preferred_element_type=jnp.float32,
        )
        sq = x_sq[:, None] + w_sq_ref[pl.ds(c * bk, bk)][None, :] + dotneg
        iota = lax.broadcasted_iota(jnp.int32, (bm, bk), 1)
        cand = jnp.min(jnp.where(sq <= hi[:, None], iota, n_k * bk),
                       axis=1) + c * bk
        best = cand if c == 0 else jnp.minimum(best, cand)
    tok_ref[0, 0, :] = best


def _tc_stage(kernel_body, extra_vec_inputs, out_dtype, flat_x, w2, x_sq,
              w_sq, bm, bk):
    m_total, d = flat_x.shape
    k_total = w2.shape[0]
    n_m = m_total // bm
    n_k = k_total // bk
    vec_specs = [pl.BlockSpec((bm,), lambda m: (m,))
                 for _ in range(len(extra_vec_inputs))]
    out3 = pl.pallas_call(
        functools.partial(kernel_body, bm, bk, n_k),
        grid=(n_m,),
        in_specs=[
            pl.BlockSpec((bm,), lambda m: (m,)),
            pl.BlockSpec((k_total,), lambda m: (0,)),
            *vec_specs,
            pl.BlockSpec((bm, d), lambda m: (m, 0)),
            pl.BlockSpec((k_total, d), lambda m: (0, 0)),
        ],
        out_specs=pl.BlockSpec((1, 1, bm), lambda m: (m, 0, 0)),
        out_shape=jax.ShapeDtypeStruct((n_m, 1, bm), out_dtype),
        compiler_params=pltpu.CompilerParams(
            vmem_limit_bytes=100 * 1024 * 1024,
        ),
    )(x_sq, w_sq, *extra_vec_inputs, flat_x, w2)
    return out3.reshape(m_total)


def _sc_gather(table, idx):
    v, d = table.shape
    m_total = idx.shape[0]
    nw = 32
    b_per_w = m_total // nw
    chunk = 192
    n_chunks = b_per_w // chunk
    mesh = plsc.VectorSubcoreMesh(core_axis_name="c", subcore_axis_name="s")

    @functools.partial(
        pl.kernel,
        mesh=mesh,
        out_type=jax.ShapeDtypeStruct((m_total, d), jnp.float32),
        scratch_types=[
            pltpu.VMEM((chunk,), jnp.int32),
            pltpu.VMEM((chunk, d), jnp.float32),
            pltpu.SemaphoreType.DMA,
        ],
    )
    def gather_kernel(table_hbm, idx_hbm, out_hbm, idx_v, rows_v, sem):
        wid = lax.axis_index("s") * 2 + lax.axis_index("c")
        base = wid * b_per_w
        for j in range(n_chunks):
            off = base + j * chunk
            pltpu.sync_copy(idx_hbm.at[pl.ds(off, chunk)], idx_v)
            pltpu.async_copy(table_hbm.at[idx_v], rows_v, sem).wait()
            pltpu.sync_copy(rows_v, out_hbm.at[pl.ds(off, chunk)])

    return gather_kernel(table, idx)


def kernel(x, w):
    b, n, d = x.shape
    m_total = b * n
    flat_x = x.reshape(-1, d)
    x_sq = jnp.sum(flat_x * flat_x, axis=1)
    w_sq = jnp.sum(w * w, axis=1)
    w2 = -2.0 * w

    min_sq = _tc_stage(_minsq_kernel, [], jnp.float32,
                       flat_x, w2, x_sq, w_sq, bm=1024, bk=2048)
    best_d = jnp.sqrt(jnp.maximum(min_sq, 0.0))
    v0i = lax.bitcast_convert_type(best_d * best_d, jnp.int32)
    hi = None
    for koff in range(-4, 5):
        vk = lax.bitcast_convert_type(jnp.maximum(v0i + koff, 0),
                                      jnp.float32)
        ok = jnp.sqrt(jnp.maximum(vk, 0.0)) == best_d
        cand = jnp.where(ok, vk, -jnp.inf)
        hi = cand if hi is None else jnp.maximum(hi, cand)

    tokens_flat = _tc_stage(_argbelow_kernel, [hi], jnp.int32,
                            flat_x, w2, x_sq, w_sq, bm=1024, bk=2048)
    vq_loss = jnp.sum(best_d * best_d) * ((1.0 + _BETA) / (m_total * d))

    quantized = _sc_gather(w, tokens_flat).reshape(b, n, d)
    tokens = tokens_flat.reshape(b, n)
    quantized_st = x + (quantized - x)
    return (tokens, quantized_st, vq_loss)
